# Initial kernel scaffold; baseline (speedup 1.0000x reference)
import jax
import jax.numpy as jnp
from jax.experimental import pallas as pl


def kernel(x, edge_index, W_ih, W_hh, b_ih, b_hh, W_gat, att_src, att_dst, gat_bias, gamma, W_lin, b_lin):
    raise NotImplementedError("write your pallas kernel here")


# speedup vs baseline: 41.5152x; 41.5152x over previous
import functools

import jax
import jax.numpy as jnp
from jax import lax
from jax.experimental import pallas as pl
from jax.experimental.pallas import tpu as pltpu
from jax.experimental.pallas import tpu_sc as plsc

NN = 50000
TT = 12
CC = 2
NPAD = 50048
SENT = NN
L = 125
NCH = 400
WARM = 64
S = L + WARM

EC = 128
NWORK = 32
E_TOTAL = 1600000 + NN
M_CHUNK = -(-E_TOTAL // (NWORK * EC))
EPAD = NWORK * EC * M_CHUNK
EW = M_CHUNK * EC

BM = NPAD // 8


def _scan_body(x0_ref, x1_ref, wih_ref, bih_ref, whh_ref, bhh_ref, gru_ref):
    w00 = wih_ref[0, 0]; w01 = wih_ref[0, 1]
    w10 = wih_ref[1, 0]; w11 = wih_ref[1, 1]
    w20 = wih_ref[2, 0]; w21 = wih_ref[2, 1]
    b0 = bih_ref[0, 0]; b1 = bih_ref[0, 1]; b2 = bih_ref[0, 2]
    u0 = whh_ref[0, 0]; u1 = whh_ref[0, 1]; u2 = whh_ref[0, 2]
    c0 = bhh_ref[0, 0]; c1 = bhh_ref[0, 1]; c2 = bhh_ref[0, 2]
    lane0 = lax.broadcasted_iota(jnp.int32, (TT, NCH), 1) == 0

    def step(s, h):
        xs0 = x0_ref[pl.ds(s, 1)].reshape(TT, NCH)
        xs1 = x1_ref[pl.ds(s, 1)].reshape(TT, NCH)
        r = jax.nn.sigmoid(xs0 * w00 + xs1 * w01 + b0 + h * u0 + c0)
        z = jax.nn.sigmoid(xs0 * w10 + xs1 * w11 + b1 + h * u1 + c1)
        nt = jnp.tanh(xs0 * w20 + xs1 * w21 + b2 + r * (h * u2 + c2))
        h = (1.0 - z) * nt + z * h
        h = jnp.where(lane0 & (s < WARM), 0.0, h)

        @pl.when(s >= WARM)
        def _():
            gru_ref[pl.ds(s - WARM, 1)] = h.reshape(1, TT, NCH)
        return h

    lax.fori_loop(0, S, step, jnp.zeros((TT, NCH), jnp.float32))


def _scan_call(x0, x1, wih, bih, whh, bhh):
    return pl.pallas_call(
        _scan_body,
        in_specs=[
            pl.BlockSpec(memory_space=pltpu.VMEM),
            pl.BlockSpec(memory_space=pltpu.VMEM),
            pl.BlockSpec(memory_space=pltpu.SMEM),
            pl.BlockSpec(memory_space=pltpu.SMEM),
            pl.BlockSpec(memory_space=pltpu.SMEM),
            pl.BlockSpec(memory_space=pltpu.SMEM),
        ],
        out_specs=pl.BlockSpec(memory_space=pltpu.VMEM),
        out_shape=jax.ShapeDtypeStruct((L, TT, NCH), jnp.float32),
    )(x0, x1, wih, bih, whh, bhh)


def _feat_body(x24_ref, wg_ref, hf_ref):
    col = lax.broadcasted_iota(jnp.int32, (1, 16), 1)
    hf_ref[...] = jnp.dot(x24_ref[...], wg_ref[...],
                          preferred_element_type=jnp.float32) + \
        jnp.where(col == 12, 1.0, 0.0)


def _feat_call(x24p, wg16p):
    return pl.pallas_call(
        _feat_body,
        grid=(NPAD // BM,),
        in_specs=[
            pl.BlockSpec((BM, TT * CC), lambda i: (i, 0)),
            pl.BlockSpec((TT * CC, 16), lambda i: (0, 0)),
        ],
        out_specs=pl.BlockSpec((BM, 16), lambda i: (i, 0)),
        out_shape=jax.ShapeDtypeStruct((NPAD, 16), jnp.float32),
    )(x24p, wg16p)


def _edge_body(hf_hbm, src_hbm, dst_hbm, zeros_hbm,
               acc_hbm,
               srcb, dstb, sbuf, rows_s, rows_d, acc_sh, sem_s, sem_d):
    c = lax.axis_index("c")
    s = lax.axis_index("s")
    wid = s * 2 + c
    rows_per = NPAD // 16
    pltpu.sync_copy(zeros_hbm.at[pl.ds(s * rows_per, rows_per)],
                    acc_sh.at[pl.ds(s * rows_per, rows_per)])
    plsc.subcore_barrier()

    base0 = wid * EW
    lanes = lax.iota(jnp.int32, 16)
    c13 = jnp.full((16,), 13, jnp.int32)
    c14 = jnp.full((16,), 14, jnp.int32)

    def chunk(g, carry):
        base = base0 + g * EC
        pltpu.sync_copy(src_hbm.at[pl.ds(base, EC)], srcb)
        pltpu.sync_copy(dst_hbm.at[pl.ds(base, EC)], dstb)
        cp_s = pltpu.async_copy(hf_hbm.at[srcb], rows_s, sem_s)
        cp_d = pltpu.async_copy(hf_hbm.at[dstb], rows_d, sem_d)
        cp_s.wait()
        cp_d.wait()
        for i in range(EC // 16):
            ri = lanes + (i * 16)
            asv = plsc.load_gather(rows_s, [ri, c13])
            adv = plsc.load_gather(rows_d, [ri, c14])
            al = asv + adv
            lr = jnp.where(al > 0.0, al, 0.2 * al)
            sbuf[pl.ds(i * 16, 16)] = jnp.exp(lr)
        for e in range(EC):
            se = plsc.load_gather(sbuf, [jnp.full((16,), e, jnp.int32)])
            rows_s[e, :] = rows_s[e, :] * se
        pltpu.sync_copy(rows_s, acc_sh.at[dstb], add=True)
        return carry

    lax.fori_loop(0, M_CHUNK, chunk, 0)
    plsc.subcore_barrier()
    pltpu.sync_copy(acc_sh.at[pl.ds(s * rows_per, rows_per)],
                    acc_hbm.at[pl.ds(c * NPAD + s * rows_per, rows_per)])


def _edge_call(hf16, src, dst, zeros):
    mesh = plsc.VectorSubcoreMesh(core_axis_name="c", subcore_axis_name="s")
    k = functools.partial(
        pl.kernel,
        mesh=mesh,
        compiler_params=pltpu.CompilerParams(
            needs_layout_passes=False, use_tc_tiling_on_sc=False),
        out_type=jax.ShapeDtypeStruct((2 * NPAD, 16), jnp.float32),
        scratch_types=[
            pltpu.VMEM((EC,), jnp.int32),
            pltpu.VMEM((EC,), jnp.int32),
            pltpu.VMEM((EC,), jnp.float32),
            pltpu.VMEM((EC, 16), jnp.float32),
            pltpu.VMEM((EC, 16), jnp.float32),
            pltpu.VMEM_SHARED((NPAD, 16), jnp.float32),
            pltpu.SemaphoreType.DMA,
            pltpu.SemaphoreType.DMA,
        ],
    )(_edge_body)
    return k(hf16, src, dst, zeros)


def _combine_body(acc0_ref, acc1_ref, gru_ref, gb_ref, gam_ref, wlt_ref,
                  bl_ref, out_ref):
    acc = acc0_ref[...] + acc1_ref[...]
    asum = acc[:, 12:13]
    x_gat = acc[:, 0:12] / (asum + 1e-16) + gb_ref[...]
    gam = gam_ref[...]
    x_mk = jax.nn.sigmoid(gam) * x_gat + \
        jax.nn.sigmoid(1.0 - gam) * gru_ref[...]
    out_ref[...] = jnp.dot(x_mk, wlt_ref[...],
                           preferred_element_type=jnp.float32) + bl_ref[...]


def _combine_call(acc0, acc1, gru, gb, gam, wlt, bl):
    return pl.pallas_call(
        _combine_body,
        grid=(NPAD // BM,),
        in_specs=[
            pl.BlockSpec((BM, 16), lambda i: (i, 0)),
            pl.BlockSpec((BM, 16), lambda i: (i, 0)),
            pl.BlockSpec((BM, TT), lambda i: (i, 0)),
            pl.BlockSpec((1, TT), lambda i: (0, 0)),
            pl.BlockSpec((1, TT), lambda i: (0, 0)),
            pl.BlockSpec((TT, 16), lambda i: (0, 0)),
            pl.BlockSpec((1, 16), lambda i: (0, 0)),
        ],
        out_specs=pl.BlockSpec((BM, 16), lambda i: (i, 0)),
        out_shape=jax.ShapeDtypeStruct((NPAD, 16), jnp.float32),
    )(acc0, acc1, gru, gb, gam, wlt, bl)


@jax.jit
def _run(x, edge_index, W_ih, W_hh, b_ih, b_hh, W_gat, att_src, att_dst,
         gat_bias, gamma, W_lin, b_lin):
    xf = x.reshape(NN, TT, CC)

    xfp = jnp.concatenate([jnp.zeros((WARM, TT, CC), x.dtype), xf], axis=0)
    r1 = xfp[:NN].reshape(NCH, L, TT, CC)[:, :WARM]
    r2 = xfp[WARM:WARM + NN].reshape(NCH, L, TT, CC)
    xw = jnp.concatenate([r1, r2], axis=1)
    xw = xw.transpose(1, 2, 3, 0)
    x0 = xw[:, :, 0, :]
    x1 = xw[:, :, 1, :]

    gru_raw = _scan_call(x0, x1, W_ih, b_ih[None, :], W_hh.T, b_hh[None, :])

    wg16 = jnp.concatenate(
        [W_gat.T, jnp.zeros((TT * CC, 4), jnp.float32)], axis=1)
    proj = jnp.eye(16, dtype=jnp.float32)
    proj = proj.at[0:TT, 13].set(att_src)
    proj = proj.at[0:TT, 14].set(att_dst)
    wg16p = wg16 @ proj
    x24p = jnp.concatenate(
        [xf.reshape(NN, TT * CC),
         jnp.zeros((NPAD - NN, TT * CC), jnp.float32)], axis=0)

    hf16 = _feat_call(x24p, wg16p)

    loops = jnp.arange(NN, dtype=jnp.int32)
    pad = jnp.full((EPAD - E_TOTAL,), SENT, jnp.int32)
    src = jnp.concatenate([edge_index[0], loops, pad])
    dst = jnp.concatenate([edge_index[1], loops, pad])
    zeros = jnp.zeros((NPAD, 16), jnp.float32)

    acc = _edge_call(hf16, src, dst, zeros)

    x_gru = jnp.concatenate(
        [gru_raw.transpose(2, 0, 1).reshape(NN, TT),
         jnp.zeros((NPAD - NN, TT), jnp.float32)], axis=0)
    out = _combine_call(acc[0:NPAD], acc[NPAD:2 * NPAD], x_gru,
                        gat_bias[None, :], gamma[None, :], W_lin.T,
                        b_lin[None, :])
    return out[0:NN].reshape(1, NN, 16)


def kernel(x, edge_index, W_ih, W_hh, b_ih, b_hh, W_gat, att_src, att_dst,
           gat_bias, gamma, W_lin, b_lin):
    return _run(x, edge_index, W_ih, W_hh, b_ih, b_hh, W_gat, att_src,
                att_dst, gat_bias, gamma, W_lin, b_lin)


# speedup vs baseline: 53.1738x; 1.2808x over previous
import functools

import jax
import jax.numpy as jnp
from jax import lax
from jax.experimental import pallas as pl
from jax.experimental.pallas import tpu as pltpu
from jax.experimental.pallas import tpu_sc as plsc

NN = 50000
TT = 12
CC = 2
NPAD = 50048
SENT = NN
L = 125
NCH = 400
WARM = 64
S = L + WARM

EC = 128
NWORK = 32
E_TOTAL = 1600000 + NN
M_CHUNK = 2 * (-(-E_TOTAL // (NWORK * EC * 2)))
EPAD = NWORK * EC * M_CHUNK
EW = M_CHUNK * EC
EALLOC = EPAD + EC

BM = NPAD // 8


def _scan_body(x0_ref, x1_ref, wih_ref, bih_ref, whh_ref, bhh_ref, gru_ref):
    w00 = wih_ref[0, 0]; w01 = wih_ref[0, 1]
    w10 = wih_ref[1, 0]; w11 = wih_ref[1, 1]
    w20 = wih_ref[2, 0]; w21 = wih_ref[2, 1]
    b0 = bih_ref[0, 0]; b1 = bih_ref[0, 1]; b2 = bih_ref[0, 2]
    u0 = whh_ref[0, 0]; u1 = whh_ref[0, 1]; u2 = whh_ref[0, 2]
    c0 = bhh_ref[0, 0]; c1 = bhh_ref[0, 1]; c2 = bhh_ref[0, 2]
    lane0 = lax.broadcasted_iota(jnp.int32, (TT, NCH), 1) == 0

    def step(s, h):
        xs0 = x0_ref[pl.ds(s, 1)].reshape(TT, NCH)
        xs1 = x1_ref[pl.ds(s, 1)].reshape(TT, NCH)
        r = jax.nn.sigmoid(xs0 * w00 + xs1 * w01 + b0 + h * u0 + c0)
        z = jax.nn.sigmoid(xs0 * w10 + xs1 * w11 + b1 + h * u1 + c1)
        nt = jnp.tanh(xs0 * w20 + xs1 * w21 + b2 + r * (h * u2 + c2))
        h = (1.0 - z) * nt + z * h
        h = jnp.where(lane0 & (s < WARM), 0.0, h)

        @pl.when(s >= WARM)
        def _():
            gru_ref[pl.ds(s - WARM, 1)] = h.reshape(1, TT, NCH)
        return h

    lax.fori_loop(0, S, step, jnp.zeros((TT, NCH), jnp.float32))


def _scan_call(x0, x1, wih, bih, whh, bhh):
    return pl.pallas_call(
        _scan_body,
        in_specs=[
            pl.BlockSpec(memory_space=pltpu.VMEM),
            pl.BlockSpec(memory_space=pltpu.VMEM),
            pl.BlockSpec(memory_space=pltpu.SMEM),
            pl.BlockSpec(memory_space=pltpu.SMEM),
            pl.BlockSpec(memory_space=pltpu.SMEM),
            pl.BlockSpec(memory_space=pltpu.SMEM),
        ],
        out_specs=pl.BlockSpec(memory_space=pltpu.VMEM),
        out_shape=jax.ShapeDtypeStruct((L, TT, NCH), jnp.float32),
    )(x0, x1, wih, bih, whh, bhh)


def _feat_body(x24_ref, wg_ref, hf_ref):
    col = lax.broadcasted_iota(jnp.int32, (1, 16), 1)
    hf_ref[...] = jnp.dot(x24_ref[...], wg_ref[...],
                          preferred_element_type=jnp.float32) + \
        jnp.where(col == 12, 1.0, 0.0)


def _feat_call(x24p, wg16p):
    return pl.pallas_call(
        _feat_body,
        grid=(NPAD // BM,),
        in_specs=[
            pl.BlockSpec((BM, TT * CC), lambda i: (i, 0)),
            pl.BlockSpec((TT * CC, 16), lambda i: (0, 0)),
        ],
        out_specs=pl.BlockSpec((BM, 16), lambda i: (i, 0)),
        out_shape=jax.ShapeDtypeStruct((NPAD, 16), jnp.float32),
    )(x24p, wg16p)


def _edge_body(hf_hbm, src_hbm, dst_hbm, zeros_hbm,
               acc_hbm,
               srcb_a, dstb_a, srcb_b, dstb_b, sbuf,
               rows_sa, rows_da, rows_sb, rows_db, acc_sh,
               sem_sa, sem_da, sem_sb, sem_db):
    c = lax.axis_index("c")
    s = lax.axis_index("s")
    wid = s * 2 + c
    rows_per = NPAD // 16
    pltpu.sync_copy(zeros_hbm.at[pl.ds(s * rows_per, rows_per)],
                    acc_sh.at[pl.ds(s * rows_per, rows_per)])
    plsc.subcore_barrier()

    base0 = wid * EW
    lanes = lax.iota(jnp.int32, 16)
    c13 = jnp.full((16,), 13, jnp.int32)
    c14 = jnp.full((16,), 14, jnp.int32)

    def prefetch(base, srcb, dstb, rows_s, rows_d, sem_s, sem_d):
        pltpu.sync_copy(src_hbm.at[pl.ds(base, EC)], srcb)
        pltpu.sync_copy(dst_hbm.at[pl.ds(base, EC)], dstb)
        pltpu.async_copy(hf_hbm.at[srcb], rows_s, sem_s)
        pltpu.async_copy(hf_hbm.at[dstb], rows_d, sem_d)

    def drain(rows_s, rows_d, sem_s, sem_d):
        pltpu.make_async_copy(hf_hbm.at[pl.ds(0, EC)], rows_s, sem_s).wait()
        pltpu.make_async_copy(hf_hbm.at[pl.ds(0, EC)], rows_d, sem_d).wait()

    def do_chunk(dstb, rows_s, rows_d):
        for i in range(EC // 16):
            ri = lanes + (i * 16)
            asv = plsc.load_gather(rows_s, [ri, c13])
            adv = plsc.load_gather(rows_d, [ri, c14])
            al = asv + adv
            lr = jnp.where(al > 0.0, al, 0.2 * al)
            sbuf[pl.ds(i * 16, 16)] = jnp.exp(lr)
        for e in range(EC):
            se = plsc.load_gather(sbuf, [jnp.full((16,), e, jnp.int32)])
            rows_s[e, :] = rows_s[e, :] * se
        pltpu.sync_copy(rows_s, acc_sh.at[dstb], add=True)

    prefetch(base0, srcb_a, dstb_a, rows_sa, rows_da, sem_sa, sem_da)

    def pair(g2, carry):
        a = base0 + (2 * g2) * EC
        prefetch(a + EC, srcb_b, dstb_b, rows_sb, rows_db, sem_sb, sem_db)
        drain(rows_sa, rows_da, sem_sa, sem_da)
        do_chunk(dstb_a, rows_sa, rows_da)
        prefetch(a + 2 * EC, srcb_a, dstb_a, rows_sa, rows_da, sem_sa, sem_da)
        drain(rows_sb, rows_db, sem_sb, sem_db)
        do_chunk(dstb_b, rows_sb, rows_db)
        return carry

    lax.fori_loop(0, M_CHUNK // 2, pair, 0)
    drain(rows_sa, rows_da, sem_sa, sem_da)
    plsc.subcore_barrier()
    pltpu.sync_copy(acc_sh.at[pl.ds(s * rows_per, rows_per)],
                    acc_hbm.at[pl.ds(c * NPAD + s * rows_per, rows_per)])


def _edge_call(hf16, src, dst, zeros):
    mesh = plsc.VectorSubcoreMesh(core_axis_name="c", subcore_axis_name="s")
    k = functools.partial(
        pl.kernel,
        mesh=mesh,
        compiler_params=pltpu.CompilerParams(
            needs_layout_passes=False, use_tc_tiling_on_sc=False),
        out_type=jax.ShapeDtypeStruct((2 * NPAD, 16), jnp.float32),
        scratch_types=[
            pltpu.VMEM((EC,), jnp.int32),
            pltpu.VMEM((EC,), jnp.int32),
            pltpu.VMEM((EC,), jnp.int32),
            pltpu.VMEM((EC,), jnp.int32),
            pltpu.VMEM((EC,), jnp.float32),
            pltpu.VMEM((EC, 16), jnp.float32),
            pltpu.VMEM((EC, 16), jnp.float32),
            pltpu.VMEM((EC, 16), jnp.float32),
            pltpu.VMEM((EC, 16), jnp.float32),
            pltpu.VMEM_SHARED((NPAD, 16), jnp.float32),
            pltpu.SemaphoreType.DMA,
            pltpu.SemaphoreType.DMA,
            pltpu.SemaphoreType.DMA,
            pltpu.SemaphoreType.DMA,
        ],
    )(_edge_body)
    return k(hf16, src, dst, zeros)


def _combine_body(acc0_ref, acc1_ref, gru_ref, gb_ref, gam_ref, wlt_ref,
                  bl_ref, out_ref):
    acc = acc0_ref[...] + acc1_ref[...]
    asum = acc[:, 12:13]
    x_gat = acc[:, 0:12] / (asum + 1e-16) + gb_ref[...]
    gam = gam_ref[...]
    x_mk = jax.nn.sigmoid(gam) * x_gat + \
        jax.nn.sigmoid(1.0 - gam) * gru_ref[...]
    out_ref[...] = jnp.dot(x_mk, wlt_ref[...],
                           preferred_element_type=jnp.float32) + bl_ref[...]


def _combine_call(acc0, acc1, gru, gb, gam, wlt, bl):
    return pl.pallas_call(
        _combine_body,
        grid=(NPAD // BM,),
        in_specs=[
            pl.BlockSpec((BM, 16), lambda i: (i, 0)),
            pl.BlockSpec((BM, 16), lambda i: (i, 0)),
            pl.BlockSpec((BM, TT), lambda i: (i, 0)),
            pl.BlockSpec((1, TT), lambda i: (0, 0)),
            pl.BlockSpec((1, TT), lambda i: (0, 0)),
            pl.BlockSpec((TT, 16), lambda i: (0, 0)),
            pl.BlockSpec((1, 16), lambda i: (0, 0)),
        ],
        out_specs=pl.BlockSpec((BM, 16), lambda i: (i, 0)),
        out_shape=jax.ShapeDtypeStruct((NPAD, 16), jnp.float32),
    )(acc0, acc1, gru, gb, gam, wlt, bl)


@jax.jit
def _run(x, edge_index, W_ih, W_hh, b_ih, b_hh, W_gat, att_src, att_dst,
         gat_bias, gamma, W_lin, b_lin):
    xf = x.reshape(NN, TT, CC)

    xfp = jnp.concatenate([jnp.zeros((WARM, TT, CC), x.dtype), xf], axis=0)
    r1 = xfp[:NN].reshape(NCH, L, TT, CC)[:, :WARM]
    r2 = xfp[WARM:WARM + NN].reshape(NCH, L, TT, CC)
    xw = jnp.concatenate([r1, r2], axis=1)
    xw = xw.transpose(1, 2, 3, 0)
    x0 = xw[:, :, 0, :]
    x1 = xw[:, :, 1, :]

    gru_raw = _scan_call(x0, x1, W_ih, b_ih[None, :], W_hh.T, b_hh[None, :])

    wg16 = jnp.concatenate(
        [W_gat.T, jnp.zeros((TT * CC, 4), jnp.float32)], axis=1)
    proj = jnp.eye(16, dtype=jnp.float32)
    proj = proj.at[0:TT, 13].set(att_src)
    proj = proj.at[0:TT, 14].set(att_dst)
    wg16p = wg16 @ proj
    x24p = jnp.concatenate(
        [xf.reshape(NN, TT * CC),
         jnp.zeros((NPAD - NN, TT * CC), jnp.float32)], axis=0)

    hf16 = _feat_call(x24p, wg16p)

    loops = jnp.arange(NN, dtype=jnp.int32)
    pad = jnp.full((EALLOC - E_TOTAL,), SENT, jnp.int32)
    src = jnp.concatenate([edge_index[0], loops, pad])
    dst = jnp.concatenate([edge_index[1], loops, pad])
    zeros = jnp.zeros((NPAD, 16), jnp.float32)

    acc = _edge_call(hf16, src, dst, zeros)

    x_gru = jnp.concatenate(
        [gru_raw.transpose(2, 0, 1).reshape(NN, TT),
         jnp.zeros((NPAD - NN, TT), jnp.float32)], axis=0)
    out = _combine_call(acc[0:NPAD], acc[NPAD:2 * NPAD], x_gru,
                        gat_bias[None, :], gamma[None, :], W_lin.T,
                        b_lin[None, :])
    return out[0:NN].reshape(1, NN, 16)


def kernel(x, edge_index, W_ih, W_hh, b_ih, b_hh, W_gat, att_src, att_dst,
           gat_bias, gamma, W_lin, b_lin):
    return _run(x, edge_index, W_ih, W_hh, b_ih, b_hh, W_gat, att_src,
                att_dst, gat_bias, gamma, W_lin, b_lin)
